# baseline (speedup 1.0000x reference)
import functools

import jax
import jax.numpy as jnp
from jax import lax
from jax.experimental import pallas as pl
from jax.experimental.pallas import tpu as pltpu
from jax.experimental.pallas import tpu_sc as plsc

N = 10000
NP = 10240
E = 320000
EP = 327680
D = 128
NC = 2
NS = 16
NW = NC * NS
CH = 80
CHS = 128
NBUF = 3

_mesh = functools.partial(
    plsc.VectorSubcoreMesh, core_axis_name="c", subcore_axis_name="s",
    num_cores=NC, num_subcores=NS)


def _ids():
    cid = lax.axis_index("c")
    sid = lax.axis_index("s")
    return cid, sid


def _fill_stripe(stripe_ref, acc_ref, start, size, value):
    vv = jnp.full((16,), value, jnp.float32)

    def body(i, _):
        stripe_ref[pl.ds(i * 16, 16)] = vv
        return 0

    lax.fori_loop(0, size // 16, body, 0)
    pltpu.sync_copy(stripe_ref, acc_ref.at[pl.ds(start, size)])


def _k1_body(col_hbm, ew_hbm, out_hbm, col_t, ew_t, stripe, acc, sem):
    cid, sid = _ids()
    ept = EP // NW
    nch = ept // CHS
    base = (cid * NS + sid) * ept
    pltpu.sync_copy(col_hbm.at[pl.ds(base, ept)], col_t)
    pltpu.sync_copy(ew_hbm.at[pl.ds(base, ept)], ew_t)

    stripe_n = NP // NS
    init = jnp.where(cid == 0, 1.0, 0.0)
    _fill_stripe(stripe, acc, sid * stripe_n, stripe_n, init)
    plsc.subcore_barrier()

    def issue(i, _):
        off = i * CHS
        pltpu.async_copy(ew_t.at[pl.ds(off, CHS)],
                         acc.at[col_t.at[pl.ds(off, CHS)]], sem, add=True)
        return 0

    def drain(i, _):
        pltpu.make_async_copy(ew_t.at[pl.ds(0, CHS)],
                              acc.at[pl.ds(0, CHS)], sem).wait()
        return 0

    lax.fori_loop(0, nch, issue, 0)
    lax.fori_loop(0, nch, drain, 0)
    plsc.subcore_barrier()
    pltpu.sync_copy(acc.at[pl.ds(sid * stripe_n, stripe_n)],
                    out_hbm.at[cid, pl.ds(sid * stripe_n, stripe_n)])


def _k1(col, ew):
    return pl.kernel(
        _k1_body,
        out_type=jax.ShapeDtypeStruct((NC, NP), jnp.float32),
        mesh=_mesh(),
        scratch_types=[
            pltpu.VMEM((EP // NW,), jnp.int32),
            pltpu.VMEM((EP // NW,), jnp.float32),
            pltpu.VMEM((NP // NS,), jnp.float32),
            pltpu.VMEM_SHARED((NP,), jnp.float32),
            pltpu.SemaphoreType.DMA,
        ],
    )(col, ew)


def _k2_body(x_ref, w_ref, d_ref, g1_ref, dv_ref):
    deg = d_ref[0] + d_ref[1]
    dv = jnp.where(deg > 0.0,
                   lax.rsqrt(jnp.maximum(deg, 1e-30)), 0.0)
    dv_ref[...] = dv
    h = jnp.dot(x_ref[...], w_ref[...], preferred_element_type=jnp.float32)
    g1_ref[...] = h * dv


def _k2(x_p, W1, degp):
    return pl.pallas_call(
        _k2_body,
        out_shape=[
            jax.ShapeDtypeStruct((NP, D), jnp.float32),
            jax.ShapeDtypeStruct((NP, 1), jnp.float32),
        ],
    )(x_p, W1, degp.reshape(NC, NP, 1))


def _k3_body(g1_hbm, row_hbm, col_hbm, ew_hbm, out_hbm,
             row_t, cbufs, ebufs, gbufs, acc, gsems, ssems, isems):
    cid, sid = _ids()
    ept = EP // NW
    nch = ept // CH
    base = (cid * NS + sid) * ept
    pltpu.sync_copy(row_hbm.at[pl.ds(base, ept)], row_t)

    zv = jnp.zeros((16,), jnp.float32)

    def zb(e, _):
        for j in range(D // 16):
            gbufs[0][e, pl.ds(j * 16, 16)] = zv
        return 0

    lax.fori_loop(0, CH, zb, 0)
    stripe_n = NP // NS
    for k in range(stripe_n // CH):
        pltpu.sync_copy(gbufs[0], acc.at[pl.ds(sid * stripe_n + k * CH, CH)])
    plsc.subcore_barrier()

    def issue_icopy(k, b):
        off = base + k * CH
        pltpu.async_copy(col_hbm.at[pl.ds(off, CH)], cbufs[b], isems[b])
        pltpu.async_copy(ew_hbm.at[pl.ds(off, CH)],
                         ebufs[b].at[pl.ds(0, CH)], isems[b])

    def wait_icopy(b):
        pltpu.make_async_copy(col_hbm.at[pl.ds(0, CH)], cbufs[b],
                              isems[b]).wait()
        pltpu.make_async_copy(ew_hbm.at[pl.ds(0, CH)],
                              ebufs[b].at[pl.ds(0, CH)], isems[b]).wait()

    def issue_gather(k, b):
        pltpu.async_copy(g1_hbm.at[row_t.at[pl.ds(k * CH, CH)]],
                         gbufs[b], gsems[b])

    def wait_gather(b):
        pltpu.make_async_copy(g1_hbm.at[pl.ds(0, CH)], gbufs[b],
                              gsems[b]).wait()

    def issue_scatter(b):
        pltpu.async_copy(gbufs[b], acc.at[cbufs[b]], ssems[b], add=True)

    def wait_scatter(b):
        pltpu.make_async_copy(gbufs[b], acc.at[pl.ds(0, CH)], ssems[b]).wait()

    def compute(b):
        def se(eq, _):
            for u in range(4):
                e = eq * 4 + u
                ev = ebufs[b][pl.ds(e, 16)]
                cv = jnp.full((16,), ev[0], jnp.float32)
                for j in range(D // 16):
                    gbufs[b][e, pl.ds(j * 16, 16)] = (
                        gbufs[b][e, pl.ds(j * 16, 16)] * cv)
            return 0

        lax.fori_loop(0, CH // 4, se, 0)

    for b in range(NBUF - 1):
        issue_icopy(b, b)
        issue_gather(b, b)

    def slot(k, _):
        for b in range(NBUF):
            @pl.when(k % NBUF == b)
            def _():
                br = (b + NBUF - 1) % NBUF
                wait_gather(b)
                wait_icopy(b)
                compute(b)
                issue_scatter(b)

                @pl.when(k + NBUF - 1 <= nch - 1)
                def _():
                    @pl.when(k >= 1)
                    def _():
                        wait_scatter(br)
                    issue_icopy(k + NBUF - 1, br)
                    issue_gather(k + NBUF - 1, br)
        return 0

    lax.fori_loop(0, nch, slot, 0)
    for b in range(NBUF):
        wait_scatter(b)
    plsc.subcore_barrier()
    pltpu.sync_copy(acc.at[pl.ds(sid * stripe_n, stripe_n)],
                    out_hbm.at[cid, pl.ds(sid * stripe_n, stripe_n)])


def _k3(g1, row, col, ew):
    return pl.kernel(
        _k3_body,
        out_type=jax.ShapeDtypeStruct((NC, NP, D), jnp.float32),
        mesh=_mesh(),
        scratch_types=[
            pltpu.VMEM((EP // NW,), jnp.int32),
            tuple(pltpu.VMEM((CH,), jnp.int32) for _ in range(NBUF)),
            tuple(pltpu.VMEM((CH + 16,), jnp.float32) for _ in range(NBUF)),
            tuple(pltpu.VMEM((CH, D), jnp.float32) for _ in range(NBUF)),
            pltpu.VMEM_SHARED((NP, D), jnp.float32),
            tuple(pltpu.SemaphoreType.DMA for _ in range(NBUF)),
            tuple(pltpu.SemaphoreType.DMA for _ in range(NBUF)),
            tuple(pltpu.SemaphoreType.DMA for _ in range(NBUF)),
        ],
    )(g1, row, col, ew)


def _k4_body(t_ref, g1_ref, dv_ref, w2_ref, b1_ref, m2_ref):
    t = t_ref[0] + t_ref[1] + g1_ref[...]
    a1 = jnp.maximum(dv_ref[...] * t + b1_ref[...], 0.0)
    h2 = jnp.sum(a1 * w2_ref[...], axis=1, keepdims=True)
    m2_ref[...] = dv_ref[...] * h2


def _k4(t1p, g1, dv, W2, b1):
    return pl.pallas_call(
        _k4_body,
        out_shape=jax.ShapeDtypeStruct((NP, 1), jnp.float32),
    )(t1p, g1, dv, W2.reshape(1, D), b1.reshape(1, D))


def _k5_body(m2_hbm, row_hbm, col_hbm, ew_hbm, out_hbm,
             row_t, col_t, ew_t, ubuf, zbuf, acc, gsem, ssem):
    cid, sid = _ids()
    ept = EP // NW
    nch = ept // CHS
    base = (cid * NS + sid) * ept
    pltpu.sync_copy(row_hbm.at[pl.ds(base, ept)], row_t)
    pltpu.sync_copy(col_hbm.at[pl.ds(base, ept)], col_t)
    pltpu.sync_copy(ew_hbm.at[pl.ds(base, ept)], ew_t)

    stripe_n = NP // NS
    _fill_stripe(zbuf, acc, sid * stripe_n, stripe_n, 0.0)
    plsc.subcore_barrier()

    def ig(k, _):
        off = k * CHS
        pltpu.async_copy(m2_hbm.at[row_t.at[pl.ds(off, CHS)]],
                         ubuf.at[pl.ds(off, CHS)], gsem)
        return 0

    def wg(k, _):
        pltpu.make_async_copy(m2_hbm.at[pl.ds(0, CHS)],
                              ubuf.at[pl.ds(0, CHS)], gsem).wait()
        return 0

    lax.fori_loop(0, nch, ig, 0)
    lax.fori_loop(0, nch, wg, 0)

    def scale(q, _):
        for u in range(4):
            g = q * 4 + u
            ubuf[pl.ds(g * 16, 16)] = (
                ubuf[pl.ds(g * 16, 16)] * ew_t[pl.ds(g * 16, 16)])
        return 0

    lax.fori_loop(0, ept // 64, scale, 0)

    def isc(k, _):
        off = k * CHS
        pltpu.async_copy(ubuf.at[pl.ds(off, CHS)],
                         acc.at[col_t.at[pl.ds(off, CHS)]], ssem, add=True)
        return 0

    def wsc(k, _):
        pltpu.make_async_copy(ubuf.at[pl.ds(0, CHS)],
                              acc.at[pl.ds(0, CHS)], ssem).wait()
        return 0

    lax.fori_loop(0, nch, isc, 0)
    lax.fori_loop(0, nch, wsc, 0)
    plsc.subcore_barrier()
    pltpu.sync_copy(acc.at[pl.ds(sid * stripe_n, stripe_n)],
                    out_hbm.at[cid, pl.ds(sid * stripe_n, stripe_n)])


def _k5(m2, row, col, ew):
    return pl.kernel(
        _k5_body,
        out_type=jax.ShapeDtypeStruct((NC, NP), jnp.float32),
        mesh=_mesh(),
        scratch_types=[
            pltpu.VMEM((EP // NW,), jnp.int32),
            pltpu.VMEM((EP // NW,), jnp.int32),
            pltpu.VMEM((EP // NW,), jnp.float32),
            pltpu.VMEM((EP // NW,), jnp.float32),
            pltpu.VMEM((NP // NS,), jnp.float32),
            pltpu.VMEM_SHARED((NP,), jnp.float32),
            pltpu.SemaphoreType.DMA,
            pltpu.SemaphoreType.DMA,
        ],
    )(m2, row, col, ew)


def _k6_body(t_ref, m2_ref, dv_ref, b2_ref, o_ref):
    t = t_ref[0] + t_ref[1] + m2_ref[...]
    o_ref[...] = dv_ref[...] * t + b2_ref[0, 0]


def _k6(t2p, m2, dv, b2):
    return pl.pallas_call(
        _k6_body,
        out_shape=jax.ShapeDtypeStruct((NP, 1), jnp.float32),
    )(t2p.reshape(NC, NP, 1), m2, dv, jnp.reshape(b2, (1, 1)))


def kernel(x, edge_index, edge_weight, W1, b1, W2, b2):
    row = jnp.pad(edge_index[0], (0, EP - E))
    col = jnp.pad(edge_index[1], (0, EP - E))
    edge_weight = jnp.pad(edge_weight, (0, EP - E))
    x_p = jnp.pad(x, ((0, NP - N), (0, 0)))

    degp = _k1(col, edge_weight)
    g1, dv = _k2(x_p, W1, degp)
    t1p = _k3(g1, row, col, edge_weight)
    m2 = _k4(t1p, g1, dv, W2, b1)
    t2p = _k5(m2.reshape(NP), row, col, edge_weight)
    outp = _k6(t2p, m2, dv, b2)
    return outp[:N].reshape(N, 1)


# speedup vs baseline: 1.9912x; 1.9912x over previous
import functools

import jax
import jax.numpy as jnp
from jax import lax
from jax.experimental import pallas as pl
from jax.experimental.pallas import tpu as pltpu
from jax.experimental.pallas import tpu_sc as plsc

N = 10000
NP = 10240
E = 320000
D = 128
NC = 2
NS = 16
NW = NC * NS
CH = 80
NBUF = 3

_mesh = functools.partial(
    plsc.VectorSubcoreMesh, core_axis_name="c", subcore_axis_name="s",
    num_cores=NC, num_subcores=NS)


def _ids():
    cid = lax.axis_index("c")
    sid = lax.axis_index("s")
    return cid, sid


def _fill_stripe(stripe_ref, acc_ref, start, size, value):
    vv = jnp.full((16,), value, jnp.float32)

    def body(i, _):
        stripe_ref[pl.ds(i * 16, 16)] = vv
        return 0

    lax.fori_loop(0, size // 16, body, 0)
    pltpu.sync_copy(stripe_ref, acc_ref.at[pl.ds(start, size)])


def _k1_body(col_hbm, ew_hbm, out_hbm, col_t, ew_t, cbuf, stripe, acc):
    cid, sid = _ids()
    ept = E // NW
    base = (cid * NS + sid) * ept
    pltpu.sync_copy(col_hbm.at[pl.ds(base, ept)], col_t)
    pltpu.sync_copy(ew_hbm.at[pl.ds(base, ept)], ew_t)

    stripe_n = NP // NS
    init = jnp.where(cid == 0, 1.0, 0.0)
    _fill_stripe(stripe, acc, sid * stripe_n, stripe_n, init)
    plsc.subcore_barrier()

    def chunk(i, _):
        off = i * CH
        for g in range(CH // 16):
            cbuf[pl.ds(g * 16, 16)] = col_t[pl.ds(off + g * 16, 16)]
        pltpu.sync_copy(ew_t.at[pl.ds(off, CH)], acc.at[cbuf], add=True)
        return 0

    lax.fori_loop(0, ept // CH, chunk, 0)
    plsc.subcore_barrier()
    pltpu.sync_copy(acc.at[pl.ds(sid * stripe_n, stripe_n)],
                    out_hbm.at[cid, pl.ds(sid * stripe_n, stripe_n)])


def _k1(col, ew):
    return pl.kernel(
        _k1_body,
        out_type=jax.ShapeDtypeStruct((NC, NP), jnp.float32),
        mesh=_mesh(),
        scratch_types=[
            pltpu.VMEM((E // NW,), jnp.int32),
            pltpu.VMEM((E // NW,), jnp.float32),
            pltpu.VMEM((CH,), jnp.int32),
            pltpu.VMEM((NP // NS,), jnp.float32),
            pltpu.VMEM_SHARED((NP,), jnp.float32),
        ],
    )(col, ew)


def _k2_body(x_ref, w_ref, d_ref, g1_ref, dv_ref):
    deg = d_ref[0] + d_ref[1]
    dv = jnp.where(deg > 0.0,
                   lax.rsqrt(jnp.maximum(deg, 1e-30)), 0.0)
    dv_ref[...] = dv
    h = jnp.dot(x_ref[...], w_ref[...], preferred_element_type=jnp.float32)
    g1_ref[...] = h * dv


def _k2(x_p, W1, degp):
    return pl.pallas_call(
        _k2_body,
        out_shape=[
            jax.ShapeDtypeStruct((NP, D), jnp.float32),
            jax.ShapeDtypeStruct((NP, 1), jnp.float32),
        ],
    )(x_p, W1, degp.reshape(NC, NP, 1))


def _k3_body(g1_hbm, row_hbm, col_hbm, ew_hbm, out_hbm,
             row_t, cbufs, ebufs, gbufs, acc, gsems, ssems, isems):
    cid, sid = _ids()
    ept = E // NW
    nch = ept // CH
    base = (cid * NS + sid) * ept
    pltpu.sync_copy(row_hbm.at[pl.ds(base, ept)], row_t)

    zv = jnp.zeros((16,), jnp.float32)

    def zb(e, _):
        for j in range(D // 16):
            gbufs[0][e, pl.ds(j * 16, 16)] = zv
        return 0

    lax.fori_loop(0, CH, zb, 0)
    stripe_n = NP // NS
    for k in range(stripe_n // CH):
        pltpu.sync_copy(gbufs[0], acc.at[pl.ds(sid * stripe_n + k * CH, CH)])
    plsc.subcore_barrier()

    def issue_icopy(k, b):
        off = base + k * CH
        pltpu.async_copy(col_hbm.at[pl.ds(off, CH)], cbufs[b], isems[b])
        pltpu.async_copy(ew_hbm.at[pl.ds(off, CH)],
                         ebufs[b].at[pl.ds(0, CH)], isems[b])

    def wait_icopy(b):
        pltpu.make_async_copy(col_hbm.at[pl.ds(0, CH)], cbufs[b],
                              isems[b]).wait()
        pltpu.make_async_copy(ew_hbm.at[pl.ds(0, CH)],
                              ebufs[b].at[pl.ds(0, CH)], isems[b]).wait()

    def issue_gather(k, b):
        pltpu.async_copy(g1_hbm.at[row_t.at[pl.ds(k * CH, CH)]],
                         gbufs[b], gsems[b])

    def wait_gather(b):
        pltpu.make_async_copy(g1_hbm.at[pl.ds(0, CH)], gbufs[b],
                              gsems[b]).wait()

    def issue_scatter(b):
        pltpu.async_copy(gbufs[b], acc.at[cbufs[b]], ssems[b], add=True)

    def wait_scatter(b):
        pltpu.make_async_copy(gbufs[b], acc.at[pl.ds(0, CH)], ssems[b]).wait()

    def compute(b):
        def se(eq, _):
            for u in range(4):
                e = eq * 4 + u
                ev = ebufs[b][pl.ds(e, 16)]
                cv = jnp.full((16,), ev[0], jnp.float32)
                for j in range(D // 16):
                    gbufs[b][e, pl.ds(j * 16, 16)] = (
                        gbufs[b][e, pl.ds(j * 16, 16)] * cv)
            return 0

        lax.fori_loop(0, CH // 4, se, 0)

    for b in range(NBUF - 1):
        issue_icopy(b, b)
        issue_gather(b, b)

    def slot(k, _):
        for b in range(NBUF):
            @pl.when(k % NBUF == b)
            def _():
                br = (b + NBUF - 1) % NBUF
                wait_gather(b)
                wait_icopy(b)
                compute(b)
                issue_scatter(b)

                @pl.when(k + NBUF - 1 <= nch - 1)
                def _():
                    @pl.when(k >= 1)
                    def _():
                        wait_scatter(br)
                    issue_icopy(k + NBUF - 1, br)
                    issue_gather(k + NBUF - 1, br)
        return 0

    lax.fori_loop(0, nch, slot, 0)
    for b in range(NBUF):
        wait_scatter(b)
    plsc.subcore_barrier()
    pltpu.sync_copy(acc.at[pl.ds(sid * stripe_n, stripe_n)],
                    out_hbm.at[cid, pl.ds(sid * stripe_n, stripe_n)])


def _k3(g1, row, col, ew):
    return pl.kernel(
        _k3_body,
        out_type=jax.ShapeDtypeStruct((NC, NP, D), jnp.float32),
        mesh=_mesh(),
        scratch_types=[
            pltpu.VMEM((E // NW,), jnp.int32),
            tuple(pltpu.VMEM((CH,), jnp.int32) for _ in range(NBUF)),
            tuple(pltpu.VMEM((CH + 16,), jnp.float32) for _ in range(NBUF)),
            tuple(pltpu.VMEM((CH, D), jnp.float32) for _ in range(NBUF)),
            pltpu.VMEM_SHARED((NP, D), jnp.float32),
            tuple(pltpu.SemaphoreType.DMA for _ in range(NBUF)),
            tuple(pltpu.SemaphoreType.DMA for _ in range(NBUF)),
            tuple(pltpu.SemaphoreType.DMA for _ in range(NBUF)),
        ],
    )(g1, row, col, ew)


def _k4_body(t_ref, g1_ref, dv_ref, w2_ref, b1_ref, m2_ref):
    t = t_ref[0] + t_ref[1] + g1_ref[...]
    a1 = jnp.maximum(dv_ref[...] * t + b1_ref[...], 0.0)
    h2 = jnp.sum(a1 * w2_ref[...], axis=1, keepdims=True)
    m2_ref[...] = dv_ref[...] * h2


def _k4(t1p, g1, dv, W2, b1):
    return pl.pallas_call(
        _k4_body,
        out_shape=jax.ShapeDtypeStruct((NP, 1), jnp.float32),
    )(t1p, g1, dv, W2.reshape(1, D), b1.reshape(1, D))


def _k5_body(m2_hbm, row_hbm, col_hbm, ew_hbm, out_hbm,
             row_t, col_t, ew_t, ubufs, zbuf, acc, gsems, ssems):
    cid, sid = _ids()
    ept = E // NW
    nch = ept // CH
    base = (cid * NS + sid) * ept
    pltpu.sync_copy(row_hbm.at[pl.ds(base, ept)], row_t)
    pltpu.sync_copy(col_hbm.at[pl.ds(base, ept)], col_t)
    pltpu.sync_copy(ew_hbm.at[pl.ds(base, ept)], ew_t)

    stripe_n = NP // NS
    _fill_stripe(zbuf, acc, sid * stripe_n, stripe_n, 0.0)
    plsc.subcore_barrier()

    def issue_gather(k, b):
        pltpu.async_copy(m2_hbm.at[row_t.at[pl.ds(k * CH, CH)]],
                         ubufs[b], gsems[b])

    def wait_gather(b):
        pltpu.make_async_copy(m2_hbm.at[pl.ds(0, CH)], ubufs[b],
                              gsems[b]).wait()

    def issue_scatter(k, b):
        pltpu.async_copy(ubufs[b], acc.at[col_t.at[pl.ds(k * CH, CH)]],
                         ssems[b], add=True)

    def wait_scatter(b):
        pltpu.make_async_copy(ubufs[b], acc.at[pl.ds(0, CH)],
                              ssems[b]).wait()

    def scale(k, b):
        off = k * CH
        for g in range(CH // 16):
            ubufs[b][pl.ds(g * 16, 16)] = (
                ubufs[b][pl.ds(g * 16, 16)] * ew_t[pl.ds(off + g * 16, 16)])

    for b in range(NBUF - 1):
        issue_gather(b, b)

    def slot(k, _):
        for b in range(NBUF):
            @pl.when(k % NBUF == b)
            def _():
                br = (b + NBUF - 1) % NBUF
                wait_gather(b)
                scale(k, b)
                issue_scatter(k, b)

                @pl.when(k + NBUF - 1 <= nch - 1)
                def _():
                    @pl.when(k >= 1)
                    def _():
                        wait_scatter(br)
                    issue_gather(k + NBUF - 1, br)
        return 0

    lax.fori_loop(0, nch, slot, 0)
    for b in range(NBUF):
        wait_scatter(b)
    plsc.subcore_barrier()
    pltpu.sync_copy(acc.at[pl.ds(sid * stripe_n, stripe_n)],
                    out_hbm.at[cid, pl.ds(sid * stripe_n, stripe_n)])


def _k5(m2, row, col, ew):
    return pl.kernel(
        _k5_body,
        out_type=jax.ShapeDtypeStruct((NC, NP), jnp.float32),
        mesh=_mesh(),
        scratch_types=[
            pltpu.VMEM((E // NW,), jnp.int32),
            pltpu.VMEM((E // NW,), jnp.int32),
            pltpu.VMEM((E // NW,), jnp.float32),
            tuple(pltpu.VMEM((CH,), jnp.float32) for _ in range(NBUF)),
            pltpu.VMEM((NP // NS,), jnp.float32),
            pltpu.VMEM_SHARED((NP,), jnp.float32),
            tuple(pltpu.SemaphoreType.DMA for _ in range(NBUF)),
            tuple(pltpu.SemaphoreType.DMA for _ in range(NBUF)),
        ],
    )(m2, row, col, ew)


def _k6_body(t_ref, m2_ref, dv_ref, b2_ref, o_ref):
    t = t_ref[0] + t_ref[1] + m2_ref[...]
    o_ref[...] = dv_ref[...] * t + b2_ref[0, 0]


def _k6(t2p, m2, dv, b2):
    return pl.pallas_call(
        _k6_body,
        out_shape=jax.ShapeDtypeStruct((NP, 1), jnp.float32),
    )(t2p.reshape(NC, NP, 1), m2, dv, jnp.reshape(b2, (1, 1)))


def kernel(x, edge_index, edge_weight, W1, b1, W2, b2):
    row = edge_index[0]
    col = edge_index[1]
    x_p = jnp.pad(x, ((0, NP - N), (0, 0)))

    degp = _k1(col, edge_weight)
    g1, dv = _k2(x_p, W1, degp)
    t1p = _k3(g1, row, col, edge_weight)
    m2 = _k4(t1p, g1, dv, W2, b1)
    t2p = _k5(m2.reshape(NP), row, col, edge_weight)
    outp = _k6(t2p, m2, dv, b2)
    return outp[:N].reshape(N, 1)


# speedup vs baseline: 1.9964x; 1.0026x over previous
import functools

import jax
import jax.numpy as jnp
from jax import lax
from jax.experimental import pallas as pl
from jax.experimental.pallas import tpu as pltpu
from jax.experimental.pallas import tpu_sc as plsc

N = 10000
NP = 10240
E = 320000
D = 128
NC = 2
NS = 16
NW = NC * NS
CH = 80
NBUF = 3

_mesh = functools.partial(
    plsc.VectorSubcoreMesh, core_axis_name="c", subcore_axis_name="s",
    num_cores=NC, num_subcores=NS)


def _ids():
    cid = lax.axis_index("c")
    sid = lax.axis_index("s")
    return cid, sid


def _fill_stripe(stripe_ref, acc_ref, start, size, value):
    vv = jnp.full((16,), value, jnp.float32)

    def body(i, _):
        stripe_ref[pl.ds(i * 16, 16)] = vv
        return 0

    lax.fori_loop(0, size // 16, body, 0)
    pltpu.sync_copy(stripe_ref, acc_ref.at[pl.ds(start, size)])


def _k1_body(col_hbm, ew_hbm, out_hbm, col_t, ew_t, cbuf, stripe, acc):
    cid, sid = _ids()
    ept = E // NW
    base = (cid * NS + sid) * ept
    pltpu.sync_copy(col_hbm.at[pl.ds(base, ept)], col_t)
    pltpu.sync_copy(ew_hbm.at[pl.ds(base, ept)], ew_t)

    stripe_n = NP // NS
    init = jnp.where(cid == 0, 1.0, 0.0)
    _fill_stripe(stripe, acc, sid * stripe_n, stripe_n, init)
    plsc.subcore_barrier()

    def chunk(i, _):
        off = i * CH
        for g in range(CH // 16):
            cbuf[pl.ds(g * 16, 16)] = col_t[pl.ds(off + g * 16, 16)]
        pltpu.sync_copy(ew_t.at[pl.ds(off, CH)], acc.at[cbuf], add=True)
        return 0

    lax.fori_loop(0, ept // CH, chunk, 0)
    plsc.subcore_barrier()
    pltpu.sync_copy(acc.at[pl.ds(sid * stripe_n, stripe_n)],
                    out_hbm.at[cid, pl.ds(sid * stripe_n, stripe_n)])


def _k1(col, ew):
    return pl.kernel(
        _k1_body,
        out_type=jax.ShapeDtypeStruct((NC, NP), jnp.float32),
        mesh=_mesh(),
        scratch_types=[
            pltpu.VMEM((E // NW,), jnp.int32),
            pltpu.VMEM((E // NW,), jnp.float32),
            pltpu.VMEM((CH,), jnp.int32),
            pltpu.VMEM((NP // NS,), jnp.float32),
            pltpu.VMEM_SHARED((NP,), jnp.float32),
        ],
    )(col, ew)


def _k2a_body(x_ref, w_ref, h_ref):
    h_ref[...] = jnp.dot(x_ref[...], w_ref[...],
                         preferred_element_type=jnp.float32)


def _k2a(x_p, W1):
    return pl.pallas_call(
        _k2a_body,
        out_shape=jax.ShapeDtypeStruct((NP, D), jnp.float32),
    )(x_p, W1)


def _k2b_body(h_ref, d_ref, g1_ref, dv_ref):
    deg = d_ref[0] + d_ref[1]
    dv = jnp.where(deg > 0.0,
                   lax.rsqrt(jnp.maximum(deg, 1e-30)), 0.0)
    dv_ref[...] = dv
    g1_ref[...] = h_ref[...] * dv


def _k2b(h, degp):
    return pl.pallas_call(
        _k2b_body,
        out_shape=[
            jax.ShapeDtypeStruct((NP, D), jnp.float32),
            jax.ShapeDtypeStruct((NP, 1), jnp.float32),
        ],
    )(h, degp.reshape(NC, NP, 1))


def _k3_body(g1_hbm, row_hbm, col_hbm, ew_hbm, out_hbm,
             row_t, cbufs, ebufs, gbufs, acc, gsems, ssems, isems):
    cid, sid = _ids()
    ept = E // NW
    nch = ept // CH
    base = (cid * NS + sid) * ept
    pltpu.sync_copy(row_hbm.at[pl.ds(base, ept)], row_t)

    zv = jnp.zeros((16,), jnp.float32)

    def zb(e, _):
        for j in range(D // 16):
            gbufs[0][e, pl.ds(j * 16, 16)] = zv
        return 0

    lax.fori_loop(0, CH, zb, 0)
    stripe_n = NP // NS
    for k in range(stripe_n // CH):
        pltpu.sync_copy(gbufs[0], acc.at[pl.ds(sid * stripe_n + k * CH, CH)])
    plsc.subcore_barrier()

    def issue_icopy(k, b):
        off = base + k * CH
        pltpu.async_copy(col_hbm.at[pl.ds(off, CH)], cbufs[b], isems[b])
        pltpu.async_copy(ew_hbm.at[pl.ds(off, CH)],
                         ebufs[b].at[pl.ds(0, CH)], isems[b])

    def wait_icopy(b):
        pltpu.make_async_copy(col_hbm.at[pl.ds(0, CH)], cbufs[b],
                              isems[b]).wait()
        pltpu.make_async_copy(ew_hbm.at[pl.ds(0, CH)],
                              ebufs[b].at[pl.ds(0, CH)], isems[b]).wait()

    def issue_gather(k, b):
        pltpu.async_copy(g1_hbm.at[row_t.at[pl.ds(k * CH, CH)]],
                         gbufs[b], gsems[b])

    def wait_gather(b):
        pltpu.make_async_copy(g1_hbm.at[pl.ds(0, CH)], gbufs[b],
                              gsems[b]).wait()

    def issue_scatter(b):
        pltpu.async_copy(gbufs[b], acc.at[cbufs[b]], ssems[b], add=True)

    def wait_scatter(b):
        pltpu.make_async_copy(gbufs[b], acc.at[pl.ds(0, CH)], ssems[b]).wait()

    def compute(b):
        def se(eq, _):
            for u in range(4):
                e = eq * 4 + u
                ev = ebufs[b][pl.ds(e, 16)]
                cv = jnp.full((16,), ev[0], jnp.float32)
                for j in range(D // 16):
                    gbufs[b][e, pl.ds(j * 16, 16)] = (
                        gbufs[b][e, pl.ds(j * 16, 16)] * cv)
            return 0

        lax.fori_loop(0, CH // 4, se, 0)

    for b in range(NBUF - 1):
        issue_icopy(b, b)
        issue_gather(b, b)

    def slot(k, _):
        for b in range(NBUF):
            @pl.when(k % NBUF == b)
            def _():
                br = (b + NBUF - 1) % NBUF
                wait_gather(b)
                wait_icopy(b)
                compute(b)
                issue_scatter(b)

                @pl.when(k + NBUF - 1 <= nch - 1)
                def _():
                    @pl.when(k >= 1)
                    def _():
                        wait_scatter(br)
                    issue_icopy(k + NBUF - 1, br)
                    issue_gather(k + NBUF - 1, br)
        return 0

    lax.fori_loop(0, nch, slot, 0)
    for b in range(NBUF):
        wait_scatter(b)
    plsc.subcore_barrier()
    pltpu.sync_copy(acc.at[pl.ds(sid * stripe_n, stripe_n)],
                    out_hbm.at[cid, pl.ds(sid * stripe_n, stripe_n)])


def _k3(g1, row, col, ew):
    return pl.kernel(
        _k3_body,
        out_type=jax.ShapeDtypeStruct((NC, NP, D), jnp.float32),
        mesh=_mesh(),
        scratch_types=[
            pltpu.VMEM((E // NW,), jnp.int32),
            tuple(pltpu.VMEM((CH,), jnp.int32) for _ in range(NBUF)),
            tuple(pltpu.VMEM((CH + 16,), jnp.float32) for _ in range(NBUF)),
            tuple(pltpu.VMEM((CH, D), jnp.float32) for _ in range(NBUF)),
            pltpu.VMEM_SHARED((NP, D), jnp.float32),
            tuple(pltpu.SemaphoreType.DMA for _ in range(NBUF)),
            tuple(pltpu.SemaphoreType.DMA for _ in range(NBUF)),
            tuple(pltpu.SemaphoreType.DMA for _ in range(NBUF)),
        ],
    )(g1, row, col, ew)


def _k4_body(t_ref, g1_ref, dv_ref, w2_ref, b1_ref, m2_ref):
    t = t_ref[0] + t_ref[1] + g1_ref[...]
    a1 = jnp.maximum(dv_ref[...] * t + b1_ref[...], 0.0)
    h2 = jnp.sum(a1 * w2_ref[...], axis=1, keepdims=True)
    m2_ref[...] = dv_ref[...] * h2


def _k4(t1p, g1, dv, W2, b1):
    return pl.pallas_call(
        _k4_body,
        out_shape=jax.ShapeDtypeStruct((NP, 1), jnp.float32),
    )(t1p, g1, dv, W2.reshape(1, D), b1.reshape(1, D))


def _k5_body(m2_hbm, row_hbm, col_hbm, ew_hbm, out_hbm,
             row_t, col_t, ew_t, ubufs, zbuf, acc, gsems, ssems):
    cid, sid = _ids()
    ept = E // NW
    nch = ept // CH
    base = (cid * NS + sid) * ept
    pltpu.sync_copy(row_hbm.at[pl.ds(base, ept)], row_t)
    pltpu.sync_copy(col_hbm.at[pl.ds(base, ept)], col_t)
    pltpu.sync_copy(ew_hbm.at[pl.ds(base, ept)], ew_t)

    stripe_n = NP // NS
    _fill_stripe(zbuf, acc, sid * stripe_n, stripe_n, 0.0)
    plsc.subcore_barrier()

    def issue_gather(k, b):
        pltpu.async_copy(m2_hbm.at[row_t.at[pl.ds(k * CH, CH)]],
                         ubufs[b], gsems[b])

    def wait_gather(b):
        pltpu.make_async_copy(m2_hbm.at[pl.ds(0, CH)], ubufs[b],
                              gsems[b]).wait()

    def issue_scatter(k, b):
        pltpu.async_copy(ubufs[b], acc.at[col_t.at[pl.ds(k * CH, CH)]],
                         ssems[b], add=True)

    def wait_scatter(b):
        pltpu.make_async_copy(ubufs[b], acc.at[pl.ds(0, CH)],
                              ssems[b]).wait()

    def scale(k, b):
        off = k * CH
        for g in range(CH // 16):
            ubufs[b][pl.ds(g * 16, 16)] = (
                ubufs[b][pl.ds(g * 16, 16)] * ew_t[pl.ds(off + g * 16, 16)])

    for b in range(NBUF - 1):
        issue_gather(b, b)

    def slot(k, _):
        for b in range(NBUF):
            @pl.when(k % NBUF == b)
            def _():
                br = (b + NBUF - 1) % NBUF
                wait_gather(b)
                scale(k, b)
                issue_scatter(k, b)

                @pl.when(k + NBUF - 1 <= nch - 1)
                def _():
                    @pl.when(k >= 1)
                    def _():
                        wait_scatter(br)
                    issue_gather(k + NBUF - 1, br)
        return 0

    lax.fori_loop(0, nch, slot, 0)
    for b in range(NBUF):
        wait_scatter(b)
    plsc.subcore_barrier()
    pltpu.sync_copy(acc.at[pl.ds(sid * stripe_n, stripe_n)],
                    out_hbm.at[cid, pl.ds(sid * stripe_n, stripe_n)])


def _k5(m2, row, col, ew):
    return pl.kernel(
        _k5_body,
        out_type=jax.ShapeDtypeStruct((NC, NP), jnp.float32),
        mesh=_mesh(),
        scratch_types=[
            pltpu.VMEM((E // NW,), jnp.int32),
            pltpu.VMEM((E // NW,), jnp.int32),
            pltpu.VMEM((E // NW,), jnp.float32),
            tuple(pltpu.VMEM((CH,), jnp.float32) for _ in range(NBUF)),
            pltpu.VMEM((NP // NS,), jnp.float32),
            pltpu.VMEM_SHARED((NP,), jnp.float32),
            tuple(pltpu.SemaphoreType.DMA for _ in range(NBUF)),
            tuple(pltpu.SemaphoreType.DMA for _ in range(NBUF)),
        ],
    )(m2, row, col, ew)


def _k6_body(t_ref, m2_ref, dv_ref, b2_ref, o_ref):
    t = t_ref[0] + t_ref[1] + m2_ref[...]
    o_ref[...] = dv_ref[...] * t + b2_ref[0, 0]


def _k6(t2p, m2, dv, b2):
    return pl.pallas_call(
        _k6_body,
        out_shape=jax.ShapeDtypeStruct((NP, 1), jnp.float32),
    )(t2p.reshape(NC, NP, 1), m2, dv, jnp.reshape(b2, (1, 1)))


def kernel(x, edge_index, edge_weight, W1, b1, W2, b2):
    row = edge_index[0]
    col = edge_index[1]
    x_p = jnp.pad(x, ((0, NP - N), (0, 0)))

    h = _k2a(x_p, W1)
    degp = _k1(col, edge_weight)
    g1, dv = _k2b(h, degp)
    t1p = _k3(g1, row, col, edge_weight)
    m2 = _k4(t1p, g1, dv, W2, b1)
    t2p = _k5(m2.reshape(NP), row, col, edge_weight)
    outp = _k6(t2p, m2, dv, b2)
    return outp[:N].reshape(N, 1)


# speedup vs baseline: 2.1059x; 1.0549x over previous
import functools

import jax
import jax.numpy as jnp
from jax import lax
from jax.experimental import pallas as pl
from jax.experimental.pallas import tpu as pltpu
from jax.experimental.pallas import tpu_sc as plsc

N = 10000
NP = 10240
E = 320000
D = 128
NC = 2
NS = 16
NW = NC * NS
CH = 80
NBUF = 3
NB5 = 4

_mesh = functools.partial(
    plsc.VectorSubcoreMesh, core_axis_name="c", subcore_axis_name="s",
    num_cores=NC, num_subcores=NS)


def _ids():
    cid = lax.axis_index("c")
    sid = lax.axis_index("s")
    return cid, sid


def _fill_stripe(stripe_ref, acc_ref, start, size, value):
    vv = jnp.full((16,), value, jnp.float32)

    def body(i, _):
        stripe_ref[pl.ds(i * 16, 16)] = vv
        return 0

    lax.fori_loop(0, size // 16, body, 0)
    pltpu.sync_copy(stripe_ref, acc_ref.at[pl.ds(start, size)])


def _k1_body(col_hbm, ew_hbm, out_hbm, col_t, ew_t, cbuf, stripe, acc):
    cid, sid = _ids()
    ept = E // NW
    base = (cid * NS + sid) * ept
    pltpu.sync_copy(col_hbm.at[pl.ds(base, ept)], col_t)
    pltpu.sync_copy(ew_hbm.at[pl.ds(base, ept)], ew_t)

    stripe_n = NP // NS
    init = jnp.where(cid == 0, 1.0, 0.0)
    _fill_stripe(stripe, acc, sid * stripe_n, stripe_n, init)
    plsc.subcore_barrier()

    def chunk(i, _):
        off = i * CH
        for g in range(CH // 16):
            cbuf[pl.ds(g * 16, 16)] = col_t[pl.ds(off + g * 16, 16)]
        pltpu.sync_copy(ew_t.at[pl.ds(off, CH)], acc.at[cbuf], add=True)
        return 0

    lax.fori_loop(0, ept // CH, chunk, 0)
    plsc.subcore_barrier()
    pltpu.sync_copy(acc.at[pl.ds(sid * stripe_n, stripe_n)],
                    out_hbm.at[cid, pl.ds(sid * stripe_n, stripe_n)])


def _k1(col, ew):
    return pl.kernel(
        _k1_body,
        out_type=jax.ShapeDtypeStruct((NC, NP), jnp.float32),
        mesh=_mesh(),
        scratch_types=[
            pltpu.VMEM((E // NW,), jnp.int32),
            pltpu.VMEM((E // NW,), jnp.float32),
            pltpu.VMEM((CH,), jnp.int32),
            pltpu.VMEM((NP // NS,), jnp.float32),
            pltpu.VMEM_SHARED((NP,), jnp.float32),
        ],
    )(col, ew)


def _k2a_body(x_ref, w_ref, h_ref):
    h_ref[...] = jnp.dot(x_ref[...], w_ref[...],
                         preferred_element_type=jnp.float32)


def _k2a(x_p, W1):
    return pl.pallas_call(
        _k2a_body,
        out_shape=jax.ShapeDtypeStruct((NP, D), jnp.float32),
    )(x_p, W1)


def _k2b_body(h_ref, d_ref, g1_ref, dv_ref):
    deg = d_ref[0] + d_ref[1]
    dv = jnp.where(deg > 0.0,
                   lax.rsqrt(jnp.maximum(deg, 1e-30)), 0.0)
    dv_ref[...] = dv
    g1_ref[...] = h_ref[...] * dv


def _k2b(h, degp):
    return pl.pallas_call(
        _k2b_body,
        out_shape=[
            jax.ShapeDtypeStruct((NP, D), jnp.float32),
            jax.ShapeDtypeStruct((NP, 1), jnp.float32),
        ],
    )(h, degp.reshape(NC, NP, 1))


def _k3_body(g1_hbm, row_hbm, col_hbm, ew_hbm, out_hbm,
             row_t, cbufs, ebufs, gbufs, acc, gsems, ssems, isems):
    cid, sid = _ids()
    ept = E // NW
    nch = ept // CH
    base = (cid * NS + sid) * ept
    pltpu.sync_copy(row_hbm.at[pl.ds(base, ept)], row_t)

    zv = jnp.zeros((16,), jnp.float32)

    def zb(e, _):
        for j in range(D // 16):
            gbufs[0][e, pl.ds(j * 16, 16)] = zv
        return 0

    lax.fori_loop(0, CH, zb, 0)
    stripe_n = NP // NS
    for k in range(stripe_n // CH):
        pltpu.sync_copy(gbufs[0], acc.at[pl.ds(sid * stripe_n + k * CH, CH)])
    plsc.subcore_barrier()

    def issue_icopy(k, b):
        off = base + k * CH
        pltpu.async_copy(col_hbm.at[pl.ds(off, CH)], cbufs[b], isems[b])
        pltpu.async_copy(ew_hbm.at[pl.ds(off, CH)],
                         ebufs[b].at[pl.ds(0, CH)], isems[b])

    def wait_icopy(b):
        pltpu.make_async_copy(col_hbm.at[pl.ds(0, CH)], cbufs[b],
                              isems[b]).wait()
        pltpu.make_async_copy(ew_hbm.at[pl.ds(0, CH)],
                              ebufs[b].at[pl.ds(0, CH)], isems[b]).wait()

    def issue_gather(k, b):
        pltpu.async_copy(g1_hbm.at[row_t.at[pl.ds(k * CH, CH)]],
                         gbufs[b], gsems[b])

    def wait_gather(b):
        pltpu.make_async_copy(g1_hbm.at[pl.ds(0, CH)], gbufs[b],
                              gsems[b]).wait()

    def issue_scatter(b):
        pltpu.async_copy(gbufs[b], acc.at[cbufs[b]], ssems[b], add=True)

    def wait_scatter(b):
        pltpu.make_async_copy(gbufs[b], acc.at[pl.ds(0, CH)], ssems[b]).wait()

    def compute(b):
        def se(eq, _):
            for u in range(4):
                e = eq * 4 + u
                ev = ebufs[b][pl.ds(e, 16)]
                cv = jnp.full((16,), ev[0], jnp.float32)
                for j in range(D // 16):
                    gbufs[b][e, pl.ds(j * 16, 16)] = (
                        gbufs[b][e, pl.ds(j * 16, 16)] * cv)
            return 0

        lax.fori_loop(0, CH // 4, se, 0)

    for b in range(NBUF - 1):
        issue_icopy(b, b)
        issue_gather(b, b)

    def slot(k, _):
        for b in range(NBUF):
            @pl.when(k % NBUF == b)
            def _():
                br = (b + NBUF - 1) % NBUF
                wait_gather(b)
                wait_icopy(b)
                compute(b)
                issue_scatter(b)

                @pl.when(k + NBUF - 1 <= nch - 1)
                def _():
                    @pl.when(k >= 1)
                    def _():
                        wait_scatter(br)
                    issue_icopy(k + NBUF - 1, br)
                    issue_gather(k + NBUF - 1, br)
        return 0

    lax.fori_loop(0, nch, slot, 0)
    for b in range(NBUF):
        wait_scatter(b)
    plsc.subcore_barrier()
    pltpu.sync_copy(acc.at[pl.ds(sid * stripe_n, stripe_n)],
                    out_hbm.at[cid, pl.ds(sid * stripe_n, stripe_n)])


def _k3(g1, row, col, ew):
    return pl.kernel(
        _k3_body,
        out_type=jax.ShapeDtypeStruct((NC, NP, D), jnp.float32),
        mesh=_mesh(),
        scratch_types=[
            pltpu.VMEM((E // NW,), jnp.int32),
            tuple(pltpu.VMEM((CH,), jnp.int32) for _ in range(NBUF)),
            tuple(pltpu.VMEM((CH + 16,), jnp.float32) for _ in range(NBUF)),
            tuple(pltpu.VMEM((CH, D), jnp.float32) for _ in range(NBUF)),
            pltpu.VMEM_SHARED((NP, D), jnp.float32),
            tuple(pltpu.SemaphoreType.DMA for _ in range(NBUF)),
            tuple(pltpu.SemaphoreType.DMA for _ in range(NBUF)),
            tuple(pltpu.SemaphoreType.DMA for _ in range(NBUF)),
        ],
    )(g1, row, col, ew)


def _k4_body(t_ref, g1_ref, dv_ref, w2_ref, b1_ref, m2_ref):
    t = t_ref[0] + t_ref[1] + g1_ref[...]
    a1 = jnp.maximum(dv_ref[...] * t + b1_ref[...], 0.0)
    h2 = jnp.sum(a1 * w2_ref[...], axis=1, keepdims=True)
    m2_ref[...] = dv_ref[...] * h2


def _k4(t1p, g1, dv, W2, b1):
    return pl.pallas_call(
        _k4_body,
        out_shape=jax.ShapeDtypeStruct((NP, 1), jnp.float32),
    )(t1p, g1, dv, W2.reshape(1, D), b1.reshape(1, D))


def _k5_body(m2_hbm, row_hbm, col_hbm, ew_hbm, out_hbm,
             row_t, col_t, ew_t, ubufs, zbuf, acc, gsems, ssems):
    cid, sid = _ids()
    ept = E // NW
    nch = ept // CH
    base = (cid * NS + sid) * ept
    pltpu.sync_copy(row_hbm.at[pl.ds(base, ept)], row_t)
    pltpu.sync_copy(col_hbm.at[pl.ds(base, ept)], col_t)
    pltpu.sync_copy(ew_hbm.at[pl.ds(base, ept)], ew_t)

    stripe_n = NP // NS
    _fill_stripe(zbuf, acc, sid * stripe_n, stripe_n, 0.0)
    plsc.subcore_barrier()

    def issue_gather(k, b):
        pltpu.async_copy(m2_hbm.at[row_t.at[pl.ds(k * CH, CH)]],
                         ubufs[b], gsems[b])

    def wait_gather(b):
        pltpu.make_async_copy(m2_hbm.at[pl.ds(0, CH)], ubufs[b],
                              gsems[b]).wait()

    def issue_scatter(k, b):
        pltpu.async_copy(ubufs[b], acc.at[col_t.at[pl.ds(k * CH, CH)]],
                         ssems[b], add=True)

    def wait_scatter(b):
        pltpu.make_async_copy(ubufs[b], acc.at[pl.ds(0, CH)],
                              ssems[b]).wait()

    def scale(k, b):
        off = k * CH
        for g in range(CH // 16):
            ubufs[b][pl.ds(g * 16, 16)] = (
                ubufs[b][pl.ds(g * 16, 16)] * ew_t[pl.ds(off + g * 16, 16)])

    for b in range(NB5 - 1):
        issue_gather(b, b)

    def slot(k, _):
        for b in range(NB5):
            @pl.when(k % NB5 == b)
            def _():
                br = (b + NB5 - 1) % NB5
                wait_gather(b)
                scale(k, b)
                issue_scatter(k, b)

                @pl.when(k + NB5 - 1 <= nch - 1)
                def _():
                    @pl.when(k >= 1)
                    def _():
                        wait_scatter(br)
                    issue_gather(k + NB5 - 1, br)
        return 0

    lax.fori_loop(0, nch, slot, 0)
    for b in range(NB5):
        wait_scatter(b)
    plsc.subcore_barrier()
    pltpu.sync_copy(acc.at[pl.ds(sid * stripe_n, stripe_n)],
                    out_hbm.at[cid, pl.ds(sid * stripe_n, stripe_n)])


def _k5(m2, row, col, ew):
    return pl.kernel(
        _k5_body,
        out_type=jax.ShapeDtypeStruct((NC, NP), jnp.float32),
        mesh=_mesh(),
        scratch_types=[
            pltpu.VMEM((E // NW,), jnp.int32),
            pltpu.VMEM((E // NW,), jnp.int32),
            pltpu.VMEM((E // NW,), jnp.float32),
            tuple(pltpu.VMEM((CH,), jnp.float32) for _ in range(NB5)),
            pltpu.VMEM((NP // NS,), jnp.float32),
            pltpu.VMEM_SHARED((NP,), jnp.float32),
            tuple(pltpu.SemaphoreType.DMA for _ in range(NB5)),
            tuple(pltpu.SemaphoreType.DMA for _ in range(NB5)),
        ],
    )(m2, row, col, ew)


def _k6_body(t_ref, m2_ref, dv_ref, b2_ref, o_ref):
    t = t_ref[0] + t_ref[1] + m2_ref[...]
    o_ref[...] = dv_ref[...] * t + b2_ref[0, 0]


def _k6(t2p, m2, dv, b2):
    return pl.pallas_call(
        _k6_body,
        out_shape=jax.ShapeDtypeStruct((NP, 1), jnp.float32),
    )(t2p.reshape(NC, NP, 1), m2, dv, jnp.reshape(b2, (1, 1)))


def kernel(x, edge_index, edge_weight, W1, b1, W2, b2):
    row = edge_index[0]
    col = edge_index[1]
    x_p = jnp.pad(x, ((0, NP - N), (0, 0)))

    h = _k2a(x_p, W1)
    degp = _k1(col, edge_weight)
    g1, dv = _k2b(h, degp)
    t1p = _k3(g1, row, col, edge_weight)
    m2 = _k4(t1p, g1, dv, W2, b1)
    t2p = _k5(m2.reshape(NP), row, col, edge_weight)
    outp = _k6(t2p, m2, dv, b2)
    return outp[:N].reshape(N, 1)


# speedup vs baseline: 2.1721x; 1.0314x over previous
import functools

import jax
import jax.numpy as jnp
from jax import lax
from jax.experimental import pallas as pl
from jax.experimental.pallas import tpu as pltpu
from jax.experimental.pallas import tpu_sc as plsc

N = 10000
NP = 10240
E = 320000
D = 128
NC = 2
NS = 16
NW = NC * NS
CH = 80
NBUF = 3
NB5 = 4

_mesh = functools.partial(
    plsc.VectorSubcoreMesh, core_axis_name="c", subcore_axis_name="s",
    num_cores=NC, num_subcores=NS)


def _ids():
    cid = lax.axis_index("c")
    sid = lax.axis_index("s")
    return cid, sid


def _fill_stripe(stripe_ref, acc_ref, start, size, value):
    vv = jnp.full((16,), value, jnp.float32)

    def body(i, _):
        stripe_ref[pl.ds(i * 16, 16)] = vv
        return 0

    lax.fori_loop(0, size // 16, body, 0)
    pltpu.sync_copy(stripe_ref, acc_ref.at[pl.ds(start, size)])


def _k1_body(col_hbm, ew_hbm, out_hbm, col_t, ew_t, stripe, acc, sems):
    cid, sid = _ids()
    ept = E // NW
    base = (cid * NS + sid) * ept
    pltpu.sync_copy(col_hbm.at[pl.ds(base, ept)], col_t)
    pltpu.sync_copy(ew_hbm.at[pl.ds(base, ept)], ew_t)

    stripe_n = NP // NS
    init = jnp.where(cid == 0, 1.0, 0.0)
    _fill_stripe(stripe, acc, sid * stripe_n, stripe_n, init)
    plsc.subcore_barrier()

    def chunk(i, _):
        for b in range(NB5):
            @pl.when(i % NB5 == b)
            def _():
                @pl.when(i >= NB5)
                def _():
                    pltpu.make_async_copy(ew_t.at[pl.ds(0, CH)],
                                          acc.at[pl.ds(0, CH)],
                                          sems[b]).wait()
                off = i * CH
                pltpu.async_copy(ew_t.at[pl.ds(off, CH)],
                                 acc.at[col_t.at[pl.ds(off, CH)]],
                                 sems[b], add=True)
        return 0

    lax.fori_loop(0, ept // CH, chunk, 0)
    for b in range(NB5):
        pltpu.make_async_copy(ew_t.at[pl.ds(0, CH)],
                              acc.at[pl.ds(0, CH)], sems[b]).wait()
    plsc.subcore_barrier()
    pltpu.sync_copy(acc.at[pl.ds(sid * stripe_n, stripe_n)],
                    out_hbm.at[cid, pl.ds(sid * stripe_n, stripe_n)])


def _k1(col, ew):
    return pl.kernel(
        _k1_body,
        out_type=jax.ShapeDtypeStruct((NC, NP), jnp.float32),
        mesh=_mesh(),
        scratch_types=[
            pltpu.VMEM((E // NW,), jnp.int32),
            pltpu.VMEM((E // NW,), jnp.float32),
            pltpu.VMEM((NP // NS,), jnp.float32),
            pltpu.VMEM_SHARED((NP,), jnp.float32),
            tuple(pltpu.SemaphoreType.DMA for _ in range(NB5)),
        ],
    )(col, ew)


def _k2a_body(x_ref, w_ref, h_ref):
    h_ref[...] = jnp.dot(x_ref[...], w_ref[...],
                         preferred_element_type=jnp.float32)


def _k2a(x_p, W1):
    return pl.pallas_call(
        _k2a_body,
        out_shape=jax.ShapeDtypeStruct((NP, D), jnp.float32),
    )(x_p, W1)


def _k2b_body(h_ref, d_ref, g1_ref, dv_ref):
    deg = d_ref[0] + d_ref[1]
    dv = jnp.where(deg > 0.0,
                   lax.rsqrt(jnp.maximum(deg, 1e-30)), 0.0)
    dv_ref[...] = dv
    g1_ref[...] = h_ref[...] * dv


def _k2b(h, degp):
    return pl.pallas_call(
        _k2b_body,
        out_shape=[
            jax.ShapeDtypeStruct((NP, D), jnp.float32),
            jax.ShapeDtypeStruct((NP, 1), jnp.float32),
        ],
    )(h, degp.reshape(NC, NP, 1))


def _k3_body(g1_hbm, row_hbm, col_hbm, ew_hbm, out_hbm,
             row_t, cbufs, ebufs, gbufs, acc, gsems, ssems, isems):
    cid, sid = _ids()
    ept = E // NW
    nch = ept // CH
    base = (cid * NS + sid) * ept
    pltpu.sync_copy(row_hbm.at[pl.ds(base, ept)], row_t)

    zv = jnp.zeros((16,), jnp.float32)

    def zb(e, _):
        for j in range(D // 16):
            gbufs[0][e, pl.ds(j * 16, 16)] = zv
        return 0

    lax.fori_loop(0, CH, zb, 0)
    stripe_n = NP // NS
    for k in range(stripe_n // CH):
        pltpu.sync_copy(gbufs[0], acc.at[pl.ds(sid * stripe_n + k * CH, CH)])
    plsc.subcore_barrier()

    def issue_icopy(k, b):
        off = base + k * CH
        pltpu.async_copy(col_hbm.at[pl.ds(off, CH)], cbufs[b], isems[b])
        pltpu.async_copy(ew_hbm.at[pl.ds(off, CH)],
                         ebufs[b].at[pl.ds(0, CH)], isems[b])

    def wait_icopy(b):
        pltpu.make_async_copy(col_hbm.at[pl.ds(0, CH)], cbufs[b],
                              isems[b]).wait()
        pltpu.make_async_copy(ew_hbm.at[pl.ds(0, CH)],
                              ebufs[b].at[pl.ds(0, CH)], isems[b]).wait()

    def issue_gather(k, b):
        pltpu.async_copy(g1_hbm.at[row_t.at[pl.ds(k * CH, CH)]],
                         gbufs[b], gsems[b])

    def wait_gather(b):
        pltpu.make_async_copy(g1_hbm.at[pl.ds(0, CH)], gbufs[b],
                              gsems[b]).wait()

    def issue_scatter(b):
        pltpu.async_copy(gbufs[b], acc.at[cbufs[b]], ssems[b], add=True)

    def wait_scatter(b):
        pltpu.make_async_copy(gbufs[b], acc.at[pl.ds(0, CH)], ssems[b]).wait()

    def compute(b):
        def se(eq, _):
            for u in range(4):
                e = eq * 4 + u
                ev = ebufs[b][pl.ds(e, 16)]
                cv = jnp.full((16,), ev[0], jnp.float32)
                for j in range(D // 16):
                    gbufs[b][e, pl.ds(j * 16, 16)] = (
                        gbufs[b][e, pl.ds(j * 16, 16)] * cv)
            return 0

        lax.fori_loop(0, CH // 4, se, 0)

    for b in range(NBUF - 1):
        issue_icopy(b, b)
        issue_gather(b, b)

    def slot(k, _):
        for b in range(NBUF):
            @pl.when(k % NBUF == b)
            def _():
                br = (b + NBUF - 1) % NBUF
                wait_gather(b)
                wait_icopy(b)
                compute(b)
                issue_scatter(b)

                @pl.when(k + NBUF - 1 <= nch - 1)
                def _():
                    @pl.when(k >= 1)
                    def _():
                        wait_scatter(br)
                    issue_icopy(k + NBUF - 1, br)
                    issue_gather(k + NBUF - 1, br)
        return 0

    lax.fori_loop(0, nch, slot, 0)
    for b in range(NBUF):
        wait_scatter(b)
    plsc.subcore_barrier()
    pltpu.sync_copy(acc.at[pl.ds(sid * stripe_n, stripe_n)],
                    out_hbm.at[cid, pl.ds(sid * stripe_n, stripe_n)])


def _k3(g1, row, col, ew):
    return pl.kernel(
        _k3_body,
        out_type=jax.ShapeDtypeStruct((NC, NP, D), jnp.float32),
        mesh=_mesh(),
        scratch_types=[
            pltpu.VMEM((E // NW,), jnp.int32),
            tuple(pltpu.VMEM((CH,), jnp.int32) for _ in range(NBUF)),
            tuple(pltpu.VMEM((CH + 16,), jnp.float32) for _ in range(NBUF)),
            tuple(pltpu.VMEM((CH, D), jnp.float32) for _ in range(NBUF)),
            pltpu.VMEM_SHARED((NP, D), jnp.float32),
            tuple(pltpu.SemaphoreType.DMA for _ in range(NBUF)),
            tuple(pltpu.SemaphoreType.DMA for _ in range(NBUF)),
            tuple(pltpu.SemaphoreType.DMA for _ in range(NBUF)),
        ],
    )(g1, row, col, ew)


def _k4_body(t_ref, g1_ref, dv_ref, w2_ref, b1_ref, m2_ref):
    t = t_ref[0] + t_ref[1] + g1_ref[...]
    a1 = jnp.maximum(dv_ref[...] * t + b1_ref[...], 0.0)
    h2 = jnp.sum(a1 * w2_ref[...], axis=1, keepdims=True)
    m2_ref[...] = dv_ref[...] * h2


def _k4(t1p, g1, dv, W2, b1):
    return pl.pallas_call(
        _k4_body,
        out_shape=jax.ShapeDtypeStruct((NP, 1), jnp.float32),
    )(t1p, g1, dv, W2.reshape(1, D), b1.reshape(1, D))


def _k5_body(m2_hbm, row_hbm, col_hbm, ew_hbm, out_hbm,
             row_t, col_t, ew_t, ubufs, zbuf, acc, gsems, ssems):
    cid, sid = _ids()
    ept = E // NW
    nch = ept // CH
    base = (cid * NS + sid) * ept
    pltpu.sync_copy(row_hbm.at[pl.ds(base, ept)], row_t)
    pltpu.sync_copy(col_hbm.at[pl.ds(base, ept)], col_t)
    pltpu.sync_copy(ew_hbm.at[pl.ds(base, ept)], ew_t)

    stripe_n = NP // NS
    _fill_stripe(zbuf, acc, sid * stripe_n, stripe_n, 0.0)
    plsc.subcore_barrier()

    def issue_gather(k, b):
        pltpu.async_copy(m2_hbm.at[row_t.at[pl.ds(k * CH, CH)]],
                         ubufs[b], gsems[b])

    def wait_gather(b):
        pltpu.make_async_copy(m2_hbm.at[pl.ds(0, CH)], ubufs[b],
                              gsems[b]).wait()

    def issue_scatter(k, b):
        pltpu.async_copy(ubufs[b], acc.at[col_t.at[pl.ds(k * CH, CH)]],
                         ssems[b], add=True)

    def wait_scatter(b):
        pltpu.make_async_copy(ubufs[b], acc.at[pl.ds(0, CH)],
                              ssems[b]).wait()

    def scale(k, b):
        off = k * CH
        for g in range(CH // 16):
            ubufs[b][pl.ds(g * 16, 16)] = (
                ubufs[b][pl.ds(g * 16, 16)] * ew_t[pl.ds(off + g * 16, 16)])

    for b in range(NB5 - 1):
        issue_gather(b, b)

    def slot(k, _):
        for b in range(NB5):
            @pl.when(k % NB5 == b)
            def _():
                br = (b + NB5 - 1) % NB5
                wait_gather(b)
                scale(k, b)
                issue_scatter(k, b)

                @pl.when(k + NB5 - 1 <= nch - 1)
                def _():
                    @pl.when(k >= 1)
                    def _():
                        wait_scatter(br)
                    issue_gather(k + NB5 - 1, br)
        return 0

    lax.fori_loop(0, nch, slot, 0)
    for b in range(NB5):
        wait_scatter(b)
    plsc.subcore_barrier()
    pltpu.sync_copy(acc.at[pl.ds(sid * stripe_n, stripe_n)],
                    out_hbm.at[cid, pl.ds(sid * stripe_n, stripe_n)])


def _k5(m2, row, col, ew):
    return pl.kernel(
        _k5_body,
        out_type=jax.ShapeDtypeStruct((NC, NP), jnp.float32),
        mesh=_mesh(),
        scratch_types=[
            pltpu.VMEM((E // NW,), jnp.int32),
            pltpu.VMEM((E // NW,), jnp.int32),
            pltpu.VMEM((E // NW,), jnp.float32),
            tuple(pltpu.VMEM((CH,), jnp.float32) for _ in range(NB5)),
            pltpu.VMEM((NP // NS,), jnp.float32),
            pltpu.VMEM_SHARED((NP,), jnp.float32),
            tuple(pltpu.SemaphoreType.DMA for _ in range(NB5)),
            tuple(pltpu.SemaphoreType.DMA for _ in range(NB5)),
        ],
    )(m2, row, col, ew)


def _k6_body(t_ref, m2_ref, dv_ref, b2_ref, o_ref):
    t = t_ref[0] + t_ref[1] + m2_ref[...]
    o_ref[...] = dv_ref[...] * t + b2_ref[0, 0]


def _k6(t2p, m2, dv, b2):
    return pl.pallas_call(
        _k6_body,
        out_shape=jax.ShapeDtypeStruct((NP, 1), jnp.float32),
    )(t2p.reshape(NC, NP, 1), m2, dv, jnp.reshape(b2, (1, 1)))


def kernel(x, edge_index, edge_weight, W1, b1, W2, b2):
    row = edge_index[0]
    col = edge_index[1]
    x_p = jnp.pad(x, ((0, NP - N), (0, 0)))

    h = _k2a(x_p, W1)
    degp = _k1(col, edge_weight)
    g1, dv = _k2b(h, degp)
    t1p = _k3(g1, row, col, edge_weight)
    m2 = _k4(t1p, g1, dv, W2, b1)
    t2p = _k5(m2.reshape(NP), row, col, edge_weight)
    outp = _k6(t2p, m2, dv, b2)
    return outp[:N].reshape(N, 1)
